# baseline (speedup 1.0000x reference)
import functools

import jax
import jax.numpy as jnp
from jax import lax
from jax.experimental import pallas as pl
from jax.experimental.pallas import tpu as pltpu
from jax.experimental.pallas import tpu_sc as plsc

NUM_CORES = 2
NUM_SUBCORES = 16
NUM_WORKERS = NUM_CORES * NUM_SUBCORES


def _make_emb_kernel(bsz, seq_len, d):
    pos_per_w = seq_len // NUM_WORKERS
    mesh = plsc.VectorSubcoreMesh(core_axis_name="c", subcore_axis_name="s")

    @functools.partial(
        pl.kernel,
        out_type=jax.ShapeDtypeStruct((bsz * seq_len, d), jnp.float32),
        mesh=mesh,
        scratch_types=[
            pltpu.VMEM((bsz * pos_per_w,), jnp.int32),
            pltpu.VMEM((bsz, pos_per_w, d), jnp.float32),
            pltpu.VMEM((pos_per_w, d), jnp.float32),
            pltpu.SemaphoreType.DMA,
            pltpu.SemaphoreType.DMA,
            pltpu.SemaphoreType.DMA,
            pltpu.SemaphoreType.DMA,
        ],
    )
    def emb(ids_hbm, tok_hbm, pos_hbm, out_hbm, idx_v, rows_v, pos_v,
            sem_i, sem_p, sem_g, sem_o):
        wid = lax.axis_index("s") * NUM_CORES + lax.axis_index("c")
        p0 = wid * pos_per_w

        idx_cps = [
            pltpu.async_copy(
                ids_hbm.at[b, pl.ds(p0, pos_per_w)],
                idx_v.at[pl.ds(b * pos_per_w, pos_per_w)],
                sem_i,
            )
            for b in range(bsz)
        ]
        pos_cp = pltpu.async_copy(pos_hbm.at[pl.ds(p0, pos_per_w)], pos_v,
                                  sem_p)
        for cp in idx_cps:
            cp.wait()

        gathers = [
            pltpu.async_copy(
                tok_hbm.at[idx_v.at[pl.ds(b * pos_per_w, pos_per_w)]],
                rows_v.at[b],
                sem_g,
            )
            for b in range(bsz)
        ]
        pos_cp.wait()

        writes = []
        for b in range(bsz):
            gathers[b].wait()

            def add_row(r, carry, b=b):
                for j in range(d // 16):
                    sl = pl.ds(j * 16, 16)
                    rows_v[b, r, sl] = rows_v[b, r, sl] + pos_v[r, sl]
                return carry

            lax.fori_loop(0, pos_per_w, add_row, 0)
            writes.append(
                pltpu.async_copy(
                    rows_v.at[b],
                    out_hbm.at[pl.ds(b * seq_len + p0, pos_per_w)],
                    sem_o,
                )
            )
        for w in writes:
            w.wait()

    return emb


def kernel(input_ids, token_table, pos_table):
    bsz, seq_len = input_ids.shape
    _, d = token_table.shape
    pos_per_w = seq_len // NUM_WORKERS
    assert seq_len % NUM_WORKERS == 0 and pos_per_w <= 128
    assert pos_per_w % 8 == 0 and d % 16 == 0

    ids = input_ids.astype(jnp.int32)
    emb = _make_emb_kernel(bsz, seq_len, d)
    out = emb(ids, token_table, pos_table)
    return out.reshape(bsz, seq_len, d)


# speedup vs baseline: 1.0041x; 1.0041x over previous
import functools

import jax
import jax.numpy as jnp
from jax import lax
from jax.experimental import pallas as pl
from jax.experimental.pallas import tpu as pltpu
from jax.experimental.pallas import tpu_sc as plsc

NUM_CORES = 2
NUM_SUBCORES = 16
NUM_WORKERS = NUM_CORES * NUM_SUBCORES
N_CHUNK = 2


def _make_emb_kernel(bsz, seq_len, d):
    pos_per_w = seq_len // NUM_WORKERS
    rows_c = pos_per_w // N_CHUNK
    mesh = plsc.VectorSubcoreMesh(core_axis_name="c", subcore_axis_name="s")

    @functools.partial(
        pl.kernel,
        out_type=jax.ShapeDtypeStruct((bsz * seq_len, d), jnp.float32),
        mesh=mesh,
        scratch_types=[
            pltpu.VMEM((bsz * pos_per_w,), jnp.int32),
            pltpu.VMEM((N_CHUNK, bsz, rows_c, d), jnp.float32),
            pltpu.VMEM((pos_per_w, d), jnp.float32),
            pltpu.SemaphoreType.DMA,
            pltpu.SemaphoreType.DMA,
            pltpu.SemaphoreType.DMA,
            pltpu.SemaphoreType.DMA,
        ],
    )
    def emb(ids_hbm, tok_hbm, pos_hbm, out_hbm, idx_v, rows_v, pos_v,
            sem_i, sem_p, sem_g, sem_o):
        wid = lax.axis_index("s") * NUM_CORES + lax.axis_index("c")
        p0 = wid * pos_per_w

        idx_cps = [
            pltpu.async_copy(
                ids_hbm.at[b, pl.ds(p0, pos_per_w)],
                idx_v.at[pl.ds(b * pos_per_w, pos_per_w)],
                sem_i,
            )
            for b in range(bsz)
        ]
        pos_cp = pltpu.async_copy(pos_hbm.at[pl.ds(p0, pos_per_w)], pos_v,
                                  sem_p)
        for cp in idx_cps:
            cp.wait()

        gathers = [
            [
                pltpu.async_copy(
                    tok_hbm.at[
                        idx_v.at[pl.ds(b * pos_per_w + h * rows_c, rows_c)]
                    ],
                    rows_v.at[h, b],
                    sem_g,
                )
                for b in range(bsz)
            ]
            for h in range(N_CHUNK)
        ]
        pos_cp.wait()

        writes = []
        for h in range(N_CHUNK):
            for cp in gathers[h]:
                cp.wait()

            def add_row(r, carry, h=h):
                pr = h * rows_c + r
                for j in range(d // 16):
                    sl = pl.ds(j * 16, 16)
                    pv = pos_v[pr, sl]
                    for b in range(bsz):
                        rows_v[h, b, r, sl] = rows_v[h, b, r, sl] + pv
                return carry

            lax.fori_loop(0, rows_c, add_row, 0)
            writes.extend(
                pltpu.async_copy(
                    rows_v.at[h, b],
                    out_hbm.at[pl.ds(b * seq_len + p0 + h * rows_c, rows_c)],
                    sem_o,
                )
                for b in range(bsz)
            )
        for w in writes:
            w.wait()

    return emb


def kernel(input_ids, token_table, pos_table):
    bsz, seq_len = input_ids.shape
    _, d = token_table.shape
    pos_per_w = seq_len // NUM_WORKERS
    assert seq_len % NUM_WORKERS == 0 and pos_per_w <= 128
    assert (pos_per_w // N_CHUNK) % 8 == 0 and d % 16 == 0

    ids = input_ids.astype(jnp.int32)
    emb = _make_emb_kernel(bsz, seq_len, d)
    out = emb(ids, token_table, pos_table)
    return out.reshape(bsz, seq_len, d)
